# Initial kernel scaffold; baseline (speedup 1.0000x reference)
import jax
import jax.numpy as jnp
from jax.experimental import pallas as pl


def kernel(indices, table):
    raise NotImplementedError("write your pallas kernel here")


# speedup vs baseline: 1.2979x; 1.2979x over previous
import functools

import jax
import jax.numpy as jnp
import numpy as np
from jax import lax
from jax.experimental import pallas as pl
from jax.experimental.pallas import tpu as pltpu
from jax.experimental.pallas import tpu_sc as plsc

DIM = 16
TRIL = DIM * (DIM + 1) // 2
OUT_ROW = DIM * DIM
B, S = 1024, 200
N = B * S

NC, NS = 2, 16
NW = NC * NS
ROWS_PER_W = N // NW
CHUNK = 128
NCHUNK = ROWS_PER_W // CHUNK

_ti, _tj = np.tril_indices(DIM)
_pos = (DIM * _ti + _tj).astype(np.int32)
_is_diag = (_ti == _tj)
_elem = np.concatenate([np.arange(128), np.arange(120, 136)])
POSMAP_NP = _pos[_elem]
FLOORS_NP = np.where(_is_diag[_elem], np.float32(1e-4),
                     np.float32(-3.0e38)).astype(np.float32)


def _tile_body(table_hbm, idx_hbm, posmap_hbm, floors_hbm, out_hbm,
               idx_v, rows_v, out_v, posmap_v, floors_v, sem):
    wid = lax.axis_index("s") * NC + lax.axis_index("c")
    base_row = wid * ROWS_PER_W

    pltpu.sync_copy(posmap_hbm, posmap_v)
    pltpu.sync_copy(floors_hbm, floors_v)

    zero16 = jnp.zeros((16,), jnp.float32)

    def zero_body(i, c):
        out_v[pl.ds(i * 16, 16)] = zero16
        return c

    lax.fori_loop(0, CHUNK * OUT_ROW // 16, zero_body, 0, unroll=4)

    pvecs = [posmap_v[pl.ds(k * 16, 16)] for k in range(9)]
    fvecs = [floors_v[pl.ds(k * 16, 16)] for k in range(9)]

    def chunk_body(g, c):
        r0 = base_row + g * CHUNK
        pltpu.sync_copy(idx_hbm.at[pl.ds(r0, CHUNK)], idx_v)
        pltpu.async_copy(table_hbm.at[idx_v], rows_v, sem).wait()

        def row_body(r, cc):
            dst = r * OUT_ROW
            for k in range(9):
                vals = rows_v[r, pl.ds((k * 16) if k < 9 - 1 else 120, 16)]
                vals = jnp.maximum(vals, fvecs[k])
                plsc.store_scatter(out_v, [pvecs[k] + dst], vals)
            return cc

        lax.fori_loop(0, CHUNK, row_body, 0)
        pltpu.sync_copy(out_v, out_hbm.at[pl.ds(r0 * OUT_ROW, CHUNK * OUT_ROW)])
        return c

    lax.fori_loop(0, NCHUNK, chunk_body, 0)


@jax.jit
def kernel(indices, table):
    idx_flat = indices.reshape(-1)
    posmap = jnp.asarray(POSMAP_NP)
    floors = jnp.asarray(FLOORS_NP)

    mesh = plsc.VectorSubcoreMesh(core_axis_name="c", subcore_axis_name="s")
    call = pl.kernel(
        _tile_body,
        mesh=mesh,
        compiler_params=pltpu.CompilerParams(
            needs_layout_passes=False, use_tc_tiling_on_sc=False),
        out_type=jax.ShapeDtypeStruct((N * OUT_ROW,), jnp.float32),
        scratch_types=[
            pltpu.VMEM((CHUNK,), jnp.int32),
            pltpu.VMEM((CHUNK, TRIL), jnp.float32),
            pltpu.VMEM((CHUNK * OUT_ROW,), jnp.float32),
            pltpu.VMEM((144,), jnp.int32),
            pltpu.VMEM((144,), jnp.float32),
            pltpu.SemaphoreType.DMA,
        ],
    )
    out_flat = call(table, idx_flat, posmap, floors)
    return out_flat.reshape(B, S, DIM, DIM)


# speedup vs baseline: 2.0500x; 1.5795x over previous
import functools

import jax
import jax.numpy as jnp
import numpy as np
from jax import lax
from jax.experimental import pallas as pl
from jax.experimental.pallas import tpu as pltpu
from jax.experimental.pallas import tpu_sc as plsc

DIM = 16
TRIL = DIM * (DIM + 1) // 2
OUT_ROW = DIM * DIM
B, S = 1024, 200
N = B * S

NC, NS = 2, 16
NW = NC * NS

VOCAB = 1000000
VPAD = 1000064
NBLK = VPAD // 128
ABLK_PER_W = -(-NBLK // NW)

CHUNK = 128
NCHUNK_TOTAL = N // CHUNK
CHUNK_PER_W = NCHUNK_TOTAL // NW

_ti, _tj = np.tril_indices(DIM)
_pos = (DIM * _ti + _tj).astype(np.int32)
_is_diag = (_ti == _tj)
_elem = np.concatenate([np.arange(128), np.arange(120, 136)])
POS128_NP = (_pos[_elem] * 128).astype(np.int32)
FLOORS_NP = np.where(_is_diag[_elem], np.float32(1e-4),
                     np.float32(-3.0e38)).astype(np.float32)


def _transpose_body(tt_hbm, rm_hbm, stage_v, rows_v):
    w = lax.axis_index("s") * NC + lax.axis_index("c")
    lane = lax.iota(jnp.int32, 16)
    rowidx = [lane + (16 * k if k < 8 else 120) for k in range(9)]

    def blk_body(i, c):
        blk = w + i * NW

        @pl.when(blk < NBLK)
        def _():
            pltpu.sync_copy(tt_hbm.at[:, pl.ds(blk * 128, 128)], stage_v)

            def ent_body(m, cc):
                mvec = jnp.broadcast_to(m, (16,)).astype(jnp.int32)
                for k in range(9):
                    vals = plsc.load_gather(stage_v, [rowidx[k], mvec])
                    off = k * 16 if k < 8 else 120
                    rows_v[pl.ds(m * TRIL + off, 16)] = vals
                return cc

            lax.fori_loop(0, 128, ent_body, 0)
            pltpu.sync_copy(rows_v, rm_hbm.at[pl.ds(blk * 128 * TRIL,
                                                    128 * TRIL)])

        return c

    lax.fori_loop(0, ABLK_PER_W, blk_body, 0)


def _expand_body(rm_hbm, idx_hbm, pos_hbm, flo_hbm, out_hbm,
                 idx_v, rows_v, out_v, pos_v, flo_v, sem_g, sem_o):
    w = lax.axis_index("s") * NC + lax.axis_index("c")
    c0 = w * CHUNK_PER_W

    pltpu.sync_copy(pos_hbm, pos_v)
    pltpu.sync_copy(flo_hbm, flo_v)

    zero16 = jnp.zeros((16,), jnp.float32)

    def zero_body(i, c):
        out_v[pl.ds(i * 16, 16)] = zero16
        return c

    lax.fori_loop(0, CHUNK * OUT_ROW // 16, zero_body, 0, unroll=4)

    pvecs = [pos_v[pl.ds(k * 16, 16)] for k in range(9)]
    fvecs = [flo_v[pl.ds(k * 16, 16)] for k in range(9)]

    def chunk_body(g, c):
        cg = c0 + g
        pltpu.sync_copy(idx_hbm.at[pl.ds(cg * CHUNK, CHUNK)], idx_v)
        pltpu.async_copy(rm_hbm.at[idx_v], rows_v, sem_g).wait()

        def row_body(r, cc):
            for k in range(9):
                vals = rows_v[r, pl.ds((k * 16) if k < 8 else 120, 16)]
                vals = jnp.maximum(vals, fvecs[k])
                plsc.store_scatter(out_v, [pvecs[k] + r], vals)
            return cc

        lax.fori_loop(0, CHUNK, row_body, 0)

        s_bb = (cg // 8) * 256 + (cg % 8)
        waits = []
        for t8 in range(32):
            waits.append(pltpu.async_copy(
                out_v.at[pl.ds(t8 * 1024, 1024)],
                out_hbm.at[pl.ds((s_bb + t8 * 8) * 1024, 1024)], sem_o))
        for h in waits:
            h.wait()
        return c

    lax.fori_loop(0, CHUNK_PER_W, chunk_body, 0)


@jax.jit
def kernel(indices, table):
    mesh = plsc.VectorSubcoreMesh(core_axis_name="c", subcore_axis_name="s")

    table_t = table.T
    transpose_call = pl.kernel(
        _transpose_body,
        mesh=mesh,
        compiler_params=pltpu.CompilerParams(
            needs_layout_passes=False, use_tc_tiling_on_sc=True,
            disable_bounds_checks=True),
        out_type=jax.ShapeDtypeStruct((VPAD * TRIL,), jnp.float32),
        scratch_types=[
            pltpu.VMEM((TRIL, 128), jnp.float32),
            pltpu.VMEM((128 * TRIL,), jnp.float32),
        ],
    )
    rm = transpose_call(table_t).reshape(VPAD, TRIL)

    idx_sT = indices.T.reshape(-1)
    pos128 = jnp.asarray(POS128_NP)
    floors = jnp.asarray(FLOORS_NP)
    expand_call = pl.kernel(
        _expand_body,
        mesh=mesh,
        compiler_params=pltpu.CompilerParams(
            needs_layout_passes=False, use_tc_tiling_on_sc=False),
        out_type=jax.ShapeDtypeStruct((N * OUT_ROW,), jnp.float32),
        scratch_types=[
            pltpu.VMEM((CHUNK,), jnp.int32),
            pltpu.VMEM((CHUNK, TRIL), jnp.float32),
            pltpu.VMEM((CHUNK * OUT_ROW,), jnp.float32),
            pltpu.VMEM((144,), jnp.int32),
            pltpu.VMEM((144,), jnp.float32),
            pltpu.SemaphoreType.DMA,
            pltpu.SemaphoreType.DMA,
        ],
    )
    out_flat = expand_call(rm, idx_sT, pos128, floors)

    t6 = out_flat.reshape(200, 16, 2, 8, 8, 128)
    return t6.transpose(3, 5, 0, 1, 2, 4).reshape(B, S, DIM, DIM)
